# Initial kernel scaffold; baseline (speedup 1.0000x reference)
import jax
import jax.numpy as jnp
from jax.experimental import pallas as pl


def kernel(weight, wgt_alpha):
    raise NotImplementedError("write your pallas kernel here")


# speedup vs baseline: 11.2806x; 11.2806x over previous
import jax
import jax.numpy as jnp
from jax.experimental import pallas as pl
from jax.experimental.pallas import tpu as pltpu

_N = 2048
_BLK = 256
_NBLK = _N // _BLK
_INV_STEP = 7.0
_NUMEL = float(_N * _N)


def _quant_kernel(alpha_ref, w_ref, o_ref, cache_ref, acc_ref):
    i = pl.program_id(0)

    @pl.when(i < _NBLK)
    def _phase1():
        x = w_ref[...]

        @pl.when(i == 0)
        def _init():
            acc_ref[0] = 0.0
            acc_ref[1] = 0.0

        acc_ref[0] += jnp.sum(x)
        acc_ref[1] += jnp.sum(x * x)
        cache_ref[pl.ds(i * _BLK, _BLK), :] = x

    @pl.when(i >= _NBLK)
    def _phase2():
        j = i - _NBLK
        mean = acc_ref[0] / _NUMEL
        var = (acc_ref[1] - _NUMEL * mean * mean) / (_NUMEL - 1.0)
        inv_salpha = 1.0 / (jnp.sqrt(var) * alpha_ref[0])
        x = cache_ref[pl.ds(j * _BLK, _BLK), :]
        xn = (x - mean) * inv_salpha
        xc = jnp.clip(xn, -1.0, 1.0)
        q = jnp.round(jnp.abs(xc) * _INV_STEP) * (1.0 / _INV_STEP)
        o_ref[...] = q * jnp.sign(xc) * alpha_ref[0]


def kernel(weight, wgt_alpha):
    alpha = jnp.reshape(wgt_alpha, (1,)).astype(jnp.float32)
    return pl.pallas_call(
        _quant_kernel,
        grid=(2 * _NBLK,),
        in_specs=[
            pl.BlockSpec(memory_space=pltpu.SMEM),
            pl.BlockSpec(
                (_BLK, _N),
                lambda i: (jnp.minimum(i, _NBLK - 1), 0),
            ),
        ],
        out_specs=pl.BlockSpec(
            (_BLK, _N),
            lambda i: (jnp.maximum(i - _NBLK, 0), 0),
        ),
        out_shape=jax.ShapeDtypeStruct((_N, _N), jnp.float32),
        scratch_shapes=[
            pltpu.VMEM((_N, _N), jnp.float32),
            pltpu.SMEM((2,), jnp.float32),
        ],
    )(alpha, weight)


# speedup vs baseline: 18.3487x; 1.6266x over previous
import jax
import jax.numpy as jnp
from jax.experimental import pallas as pl
from jax.experimental.pallas import tpu as pltpu

_N = 2048
_BLK = 512
_NBLK = _N // _BLK
_INV_STEP = 7.0
_NUMEL = float(_N * _N)


def _quant_kernel(alpha_ref, w_ref, o_ref, cache_ref, acc_ref):
    i = pl.program_id(0)

    @pl.when(i < _NBLK)
    def _phase1():
        x = w_ref[...]

        @pl.when(i == 0)
        def _init():
            acc_ref[0] = 0.0
            acc_ref[1] = 0.0

        acc_ref[0] += jnp.sum(x)
        acc_ref[1] += jnp.sum(x * x)
        cache_ref[pl.ds(i * _BLK, _BLK), :] = x

    @pl.when(i >= _NBLK)
    def _phase2():
        j = i - _NBLK
        mean = acc_ref[0] / _NUMEL
        var = (acc_ref[1] - _NUMEL * mean * mean) / (_NUMEL - 1.0)
        a = _INV_STEP / (jnp.sqrt(var) * alpha_ref[0])
        b = -mean * a
        c = alpha_ref[0] * (1.0 / _INV_STEP)
        x = cache_ref[pl.ds(j * _BLK, _BLK), :]
        xs = jnp.clip(x * a + b, -_INV_STEP, _INV_STEP)
        o_ref[...] = jnp.round(xs) * c


def kernel(weight, wgt_alpha):
    alpha = jnp.reshape(wgt_alpha, (1,)).astype(jnp.float32)
    return pl.pallas_call(
        _quant_kernel,
        grid=(2 * _NBLK,),
        in_specs=[
            pl.BlockSpec(memory_space=pltpu.SMEM),
            pl.BlockSpec(
                (_BLK, _N),
                lambda i: (jnp.minimum(i, _NBLK - 1), 0),
            ),
        ],
        out_specs=pl.BlockSpec(
            (_BLK, _N),
            lambda i: (jnp.maximum(i - _NBLK, 0), 0),
        ),
        out_shape=jax.ShapeDtypeStruct((_N, _N), jnp.float32),
        scratch_shapes=[
            pltpu.VMEM((_N, _N), jnp.float32),
            pltpu.SMEM((2,), jnp.float32),
        ],
    )(alpha, weight)
